# baseline (speedup 1.0000x reference)
import functools

import jax
import jax.numpy as jnp
from jax import lax
from jax.experimental import pallas as pl
from jax.experimental.pallas import tpu as pltpu
from jax.experimental.pallas import tpu_sc as plsc

N = 10000
E = 320000
F = 128
D = 1024

NC = 2
NS = 16
NW = NC * NS

NP = 10240
RPT = NP // NS
C = 128
ECH = 128
ER = E // ECH
ERP = 2560
CH = 8
EOP = 128
SUB = ECH // EOP
CSUB = CH * SUB
NBUF = 2
LAG = 1

BM = 1000
GM = N // BM

_mesh = plsc.VectorSubcoreMesh(
    core_axis_name="c", subcore_axis_name="s", num_cores=NC, num_subcores=NS)


def _zero_vmem(ref, rows, cols):
    zv = jnp.zeros((16,), jnp.float32)

    def body(i, _):
        for j in range(cols // 16):
            ref[i, pl.ds(j * 16, 16)] = zv
        return 0

    lax.fori_loop(0, rows, body, 0)


@functools.partial(
    pl.kernel,
    out_type=jax.ShapeDtypeStruct((NC, NP), jnp.float32),
    mesh=_mesh,
    scratch_types=[
        pltpu.VMEM((1, ECH), jnp.int32),
        pltpu.VMEM((ECH,), jnp.float32),
        pltpu.VMEM((1, RPT), jnp.float32),
        pltpu.VMEM_SHARED((NP,), jnp.float32),
    ],
)
def _sc_degree(dst_hbm, out_hbm, idx_v, ones_v, zrow_v, acc_sh):
    c = lax.axis_index("c")
    s = lax.axis_index("s")
    w = s * NC + c
    row0 = w * (ERP // NW)
    for j in range(ECH // 16):
        ones_v[pl.ds(j * 16, 16)] = jnp.ones((16,), jnp.float32)
    _zero_vmem(zrow_v, 1, RPT)
    pltpu.sync_copy(zrow_v.at[0], acc_sh.at[pl.ds(s * RPT, RPT)])
    plsc.subcore_barrier()

    def body(i, _):
        pltpu.sync_copy(dst_hbm.at[row0 + i], idx_v.at[0])
        pltpu.sync_copy(ones_v, acc_sh.at[idx_v.at[0]], add=True)
        return 0

    lax.fori_loop(0, ERP // NW, body, 0)
    plsc.subcore_barrier()
    pltpu.sync_copy(acc_sh.at[pl.ds(s * RPT, RPT)],
                    out_hbm.at[c, pl.ds(s * RPT, RPT)])


def _make_prop(NB):
    n_out = NB if NB > 1 else 2

    @functools.partial(
        pl.kernel,
        out_type=jax.ShapeDtypeStruct((n_out, NP, C), jnp.float32),
        mesh=_mesh,
        scratch_types=[
            pltpu.VMEM((CSUB, EOP), jnp.int32),
            pltpu.VMEM((CSUB, EOP), jnp.int32),
            pltpu.VMEM((CSUB, EOP), jnp.int32),
            pltpu.VMEM((CSUB, EOP), jnp.int32),
            [pltpu.VMEM((EOP, C), jnp.float32)] * NBUF,
            pltpu.VMEM((64, C), jnp.float32),
            pltpu.VMEM_SHARED((NP, C), jnp.float32),
            [pltpu.SemaphoreType.DMA] * NBUF,
            [pltpu.SemaphoreType.DMA] * NBUF,
        ],
    )
    def prop(table_hbm, src_hbm, dst_hbm, out_hbm,
             sidx_a, didx_a, sidx_b, didx_b, bufs, zrow_v, acc_sh,
             gsem, ssem):
        c = lax.axis_index("c")
        s = lax.axis_index("s")
        _zero_vmem(zrow_v, 64, C)

        if NB > 1:
            rows_per, first = ERP // NS, s * (ERP // NS)
        else:
            w = s * NC + c
            rows_per, first = ERP // NW, w * (ERP // NW)
        nchunks = rows_per // CH

        def one_block(p, out_slot):
            tbl = table_hbm.at[p]
            for j in range(RPT // 64):
                pltpu.sync_copy(
                    zrow_v, acc_sh.at[pl.ds(s * RPT + j * 64, 64)])
            plsc.subcore_barrier()

            def wait_scatter(k, didx):
                pltpu.make_async_copy(
                    bufs[k], acc_sh.at[didx.at[0]], ssem[k]).wait()

            def sub_chunk(ci, sidx, didx, base):
                pltpu.sync_copy(src_hbm.at[pl.ds(base, CSUB)], sidx)
                pltpu.sync_copy(dst_hbm.at[pl.ds(base, CSUB)], didx)
                for u in range(CSUB):
                    k = u % NBUF
                    if u < NBUF:
                        @pl.when(ci > 0)
                        def _():
                            wait_scatter(k, didx)
                    else:
                        wait_scatter(k, didx)
                    pltpu.make_async_copy(
                        tbl.at[sidx.at[u]], bufs[k], gsem[k]).start()
                    if u >= LAG:
                        v = u - LAG
                        k2 = v % NBUF
                        pltpu.make_async_copy(
                            tbl.at[sidx.at[v]], bufs[k2], gsem[k2]).wait()
                        pltpu.make_async_copy(
                            bufs[k2], acc_sh.at[didx.at[v]],
                            ssem[k2]).start(add=True)
                for v in range(CSUB - LAG, CSUB):
                    k2 = v % NBUF
                    pltpu.make_async_copy(
                        tbl.at[sidx.at[v]], bufs[k2], gsem[k2]).wait()
                    pltpu.make_async_copy(
                        bufs[k2], acc_sh.at[didx.at[v]],
                        ssem[k2]).start(add=True)

            def chunk(ci, _):
                base = (first + ci * CH) * SUB

                @pl.when(lax.rem(ci, 2) == 0)
                def _():
                    sub_chunk(ci, sidx_a, didx_a, base)

                @pl.when(lax.rem(ci, 2) == 1)
                def _():
                    sub_chunk(ci, sidx_b, didx_b, base)

                return 0

            lax.fori_loop(0, nchunks, chunk, 0)
            for k in range(NBUF):
                wait_scatter(k, didx_a)
            plsc.subcore_barrier()
            pltpu.sync_copy(acc_sh.at[pl.ds(s * RPT, RPT)],
                            out_hbm.at[out_slot, pl.ds(s * RPT, RPT)])

        if NB > 1:
            for pi in range(NB // NC):
                p = pi * NC + c
                one_block(p, p)
        else:
            one_block(0, c)

    return prop


_prop1 = _make_prop(1)
_prop8 = _make_prop(D // C)


def _norm_body(degp_ref, out_ref):
    d = degp_ref[0:1, :] + degp_ref[1:2, :]
    out_ref[...] = lax.rsqrt(jnp.maximum(d, 1.0))


def _scale_x_body(x_ref, nc_ref, out_ref):
    out_ref[...] = x_ref[...] * nc_ref[...]


def _scale_r_body(r_ref, nc_ref, out_ref):
    n2 = nc_ref[...] * nc_ref[...]
    out_ref[0] = (r_ref[0] + r_ref[1]) * n2


def _scale_q_body(q_ref, nc_ref, out_ref):
    n2 = nc_ref[...] * nc_ref[...]
    for k in range(D // C):
        out_ref[k] = q_ref[k] * n2


def _l1_body(x_ref, r1_ref, r2_ref, nc_ref, w_ref, b_ref, h_ref, hn_ref):
    nc = nc_ref[...]
    p1 = (r1_ref[0] + r1_ref[1]) * nc
    p2 = (r2_ref[0] + r2_ref[1]) * nc
    acc = jnp.dot(x_ref[...], w_ref[0:F, :],
                  preferred_element_type=jnp.float32)
    acc += jnp.dot(p1, w_ref[F:2 * F, :], preferred_element_type=jnp.float32)
    acc += jnp.dot(p2, w_ref[2 * F:3 * F, :],
                   preferred_element_type=jnp.float32)
    h = jnp.maximum(acc + b_ref[...], 0.0)
    h_ref[...] = h
    hn = h * nc
    for k in range(D // C):
        hn_ref[k] = hn[:, k * C:(k + 1) * C]


def _l2_body(h_ref, q1_ref, q2_ref, nc_ref, w_ref, b_ref, cm_ref, sn_ref):
    nc = nc_ref[...]
    acc = jnp.dot(h_ref[...], w_ref[0:D, :], preferred_element_type=jnp.float32)
    for k in range(D // C):
        acc += jnp.dot(q1_ref[k] * nc, w_ref[D + k * C:D + (k + 1) * C, :],
                       preferred_element_type=jnp.float32)
        acc += jnp.dot(q2_ref[k] * nc, w_ref[2 * D + k * C:2 * D + (k + 1) * C, :],
                       preferred_element_type=jnp.float32)
    g = acc + b_ref[...]
    cm_ref[0] = jnp.max(g, axis=0, keepdims=True)
    rn = jnp.sqrt(jnp.sum(g * g, axis=1, keepdims=True))
    sn_ref[0] = jnp.broadcast_to(jnp.sum(rn), (1, 128))


def _head_body(cm_ref, sn_ref, wo1_ref, bo1_ref, wo2_ref, bo2_ref, out_ref):
    colmax = jnp.max(cm_ref[...], axis=0)
    total = jnp.sum(sn_ref[...], axis=0)[:, 0:1]
    factor = (jnp.sqrt(jnp.float32(D)) * jnp.float32(N)) / total
    emb = colmax * factor
    z = jnp.dot(emb, wo1_ref[...], preferred_element_type=jnp.float32)
    z = z + bo1_ref[...]
    z = jnp.where(z > 0, z, 0.01 * z)
    z2 = jnp.dot(z, wo2_ref[...], preferred_element_type=jnp.float32)
    z2 = z2 + bo2_ref[...]
    m = jnp.max(z2, axis=1, keepdims=True)
    lse = jnp.log(jnp.sum(jnp.exp(z2 - m), axis=1, keepdims=True)) + m
    out_ref[...] = z2 - lse


def _row_blocks(*dims):
    def spec(shape, idx_dim):
        def imap(i):
            return tuple(i if d == idx_dim else 0 for d in range(len(shape)))
        return pl.BlockSpec(shape, imap)
    return spec


def kernel(x, edge_index, W1, b1, W2, b2, Wo1, bo1, Wo2, bo2):
    pad = (ERP - ER) * ECH
    pad_src = (jnp.arange(pad, dtype=jnp.int32) * 13) % N
    pad_dst = N + (jnp.arange(pad, dtype=jnp.int32) % (NP - N))
    src2d = jnp.concatenate(
        [edge_index[0].astype(jnp.int32), pad_src]).reshape(ERP, ECH)
    dst2d = jnp.concatenate(
        [edge_index[1].astype(jnp.int32), pad_dst]).reshape(ERP, ECH)
    srcS = src2d.reshape(ERP * SUB, EOP)
    dstS = dst2d.reshape(ERP * SUB, EOP)

    degp = _sc_degree(dst2d)

    norm_row = pl.pallas_call(
        _norm_body,
        out_shape=jax.ShapeDtypeStruct((1, NP), jnp.float32),
    )(degp)
    norm_col = norm_row.reshape(NP, 1)

    nc_spec = pl.BlockSpec((BM, 1), lambda i: (i, 0))

    xn = pl.pallas_call(
        _scale_x_body,
        grid=(GM,),
        in_specs=[pl.BlockSpec((BM, F), lambda i: (i, 0)), nc_spec],
        out_specs=pl.BlockSpec((BM, F), lambda i: (i, 0)),
        out_shape=jax.ShapeDtypeStruct((N, F), jnp.float32),
    )(x, norm_col)

    r1 = _prop1(xn.reshape(1, N, F), srcS, dstS)

    t1 = pl.pallas_call(
        _scale_r_body,
        grid=(GM,),
        in_specs=[pl.BlockSpec((2, BM, F), lambda i: (0, i, 0)), nc_spec],
        out_specs=pl.BlockSpec((1, BM, F), lambda i: (0, i, 0)),
        out_shape=jax.ShapeDtypeStruct((1, N, F), jnp.float32),
    )(r1, norm_col)

    r2 = _prop1(t1, srcS, dstS)

    h, hn = pl.pallas_call(
        _l1_body,
        grid=(GM,),
        in_specs=[
            pl.BlockSpec((BM, F), lambda i: (i, 0)),
            pl.BlockSpec((2, BM, F), lambda i: (0, i, 0)),
            pl.BlockSpec((2, BM, F), lambda i: (0, i, 0)),
            nc_spec,
            pl.BlockSpec((3 * F, D), lambda i: (0, 0)),
            pl.BlockSpec((1, D), lambda i: (0, 0)),
        ],
        out_specs=[
            pl.BlockSpec((BM, D), lambda i: (i, 0)),
            pl.BlockSpec((D // C, BM, C), lambda i: (0, i, 0)),
        ],
        out_shape=[
            jax.ShapeDtypeStruct((N, D), jnp.float32),
            jax.ShapeDtypeStruct((D // C, N, C), jnp.float32),
        ],
    )(x, r1, r2, norm_col, W1, b1.reshape(1, D))

    q1 = _prop8(hn, srcS, dstS)

    t2 = pl.pallas_call(
        _scale_q_body,
        grid=(GM,),
        in_specs=[pl.BlockSpec((D // C, BM, C), lambda i: (0, i, 0)), nc_spec],
        out_specs=pl.BlockSpec((D // C, BM, C), lambda i: (0, i, 0)),
        out_shape=jax.ShapeDtypeStruct((D // C, N, C), jnp.float32),
    )(q1, norm_col)

    q2 = _prop8(t2, srcS, dstS)

    cm, sn = pl.pallas_call(
        _l2_body,
        grid=(GM,),
        in_specs=[
            pl.BlockSpec((BM, D), lambda i: (i, 0)),
            pl.BlockSpec((D // C, BM, C), lambda i: (0, i, 0)),
            pl.BlockSpec((D // C, BM, C), lambda i: (0, i, 0)),
            nc_spec,
            pl.BlockSpec((3 * D, D), lambda i: (0, 0)),
            pl.BlockSpec((1, D), lambda i: (0, 0)),
        ],
        out_specs=[
            pl.BlockSpec((1, 1, D), lambda i: (i, 0, 0)),
            pl.BlockSpec((1, 1, 128), lambda i: (i, 0, 0)),
        ],
        out_shape=[
            jax.ShapeDtypeStruct((GM, 1, D), jnp.float32),
            jax.ShapeDtypeStruct((GM, 1, 128), jnp.float32),
        ],
    )(h, q1, q2, norm_col, W2, b2.reshape(1, D))

    out = pl.pallas_call(
        _head_body,
        out_shape=jax.ShapeDtypeStruct((1, 3), jnp.float32),
    )(cm, sn, Wo1, bo1.reshape(1, 256), Wo2, bo2.reshape(1, 3))
    return out


# speedup vs baseline: 1.1041x; 1.1041x over previous
import functools

import jax
import jax.numpy as jnp
from jax import lax
from jax.experimental import pallas as pl
from jax.experimental.pallas import tpu as pltpu
from jax.experimental.pallas import tpu_sc as plsc

N = 10000
E = 320000
F = 128
D = 1024

NC = 2
NS = 16
NW = NC * NS

NP = 10240
RPT = NP // NS
C = 128
ECH = 128
ER = E // ECH
ERP = 2560
CH = 8
EOP = 64
SUB = ECH // EOP
CSUB = CH * SUB
NBUF = 4
LAG = 2

BM = 1000
GM = N // BM

_mesh = plsc.VectorSubcoreMesh(
    core_axis_name="c", subcore_axis_name="s", num_cores=NC, num_subcores=NS)


def _zero_vmem(ref, rows, cols):
    zv = jnp.zeros((16,), jnp.float32)

    def body(i, _):
        for j in range(cols // 16):
            ref[i, pl.ds(j * 16, 16)] = zv
        return 0

    lax.fori_loop(0, rows, body, 0)


@functools.partial(
    pl.kernel,
    out_type=jax.ShapeDtypeStruct((NC, NP), jnp.float32),
    mesh=_mesh,
    scratch_types=[
        pltpu.VMEM((1, ECH), jnp.int32),
        pltpu.VMEM((ECH,), jnp.float32),
        pltpu.VMEM((1, RPT), jnp.float32),
        pltpu.VMEM_SHARED((NP,), jnp.float32),
    ],
)
def _sc_degree(dst_hbm, out_hbm, idx_v, ones_v, zrow_v, acc_sh):
    c = lax.axis_index("c")
    s = lax.axis_index("s")
    w = s * NC + c
    row0 = w * (ERP // NW)
    for j in range(ECH // 16):
        ones_v[pl.ds(j * 16, 16)] = jnp.ones((16,), jnp.float32)
    _zero_vmem(zrow_v, 1, RPT)
    pltpu.sync_copy(zrow_v.at[0], acc_sh.at[pl.ds(s * RPT, RPT)])
    plsc.subcore_barrier()

    def body(i, _):
        pltpu.sync_copy(dst_hbm.at[row0 + i], idx_v.at[0])
        pltpu.sync_copy(ones_v, acc_sh.at[idx_v.at[0]], add=True)
        return 0

    lax.fori_loop(0, ERP // NW, body, 0)
    plsc.subcore_barrier()
    pltpu.sync_copy(acc_sh.at[pl.ds(s * RPT, RPT)],
                    out_hbm.at[c, pl.ds(s * RPT, RPT)])


def _make_prop(NB):
    n_out = NB if NB > 1 else 2

    @functools.partial(
        pl.kernel,
        out_type=jax.ShapeDtypeStruct((n_out, NP, C), jnp.float32),
        mesh=_mesh,
        scratch_types=[
            pltpu.VMEM((CSUB, EOP), jnp.int32),
            pltpu.VMEM((CSUB, EOP), jnp.int32),
            pltpu.VMEM((CSUB, EOP), jnp.int32),
            pltpu.VMEM((CSUB, EOP), jnp.int32),
            [pltpu.VMEM((EOP, C), jnp.float32)] * NBUF,
            pltpu.VMEM((64, C), jnp.float32),
            pltpu.VMEM_SHARED((NP, C), jnp.float32),
            [pltpu.SemaphoreType.DMA] * NBUF,
            [pltpu.SemaphoreType.DMA] * NBUF,
            pltpu.SemaphoreType.DMA,
        ],
    )
    def prop(table_hbm, src_hbm, dst_hbm, out_hbm,
             sidx_a, didx_a, sidx_b, didx_b, bufs, zrow_v, acc_sh,
             gsem, ssem, isem):
        c = lax.axis_index("c")
        s = lax.axis_index("s")
        _zero_vmem(zrow_v, 64, C)

        if NB > 1:
            rows_per, first = ERP // NS, s * (ERP // NS)
        else:
            w = s * NC + c
            rows_per, first = ERP // NW, w * (ERP // NW)
        nchunks = rows_per // CH

        def one_block(p, out_slot):
            tbl = table_hbm.at[p]
            for j in range(RPT // 64):
                pltpu.sync_copy(
                    zrow_v, acc_sh.at[pl.ds(s * RPT + j * 64, 64)])
            plsc.subcore_barrier()

            def wait_scatter(k, didx):
                pltpu.make_async_copy(
                    bufs[k], acc_sh.at[didx.at[0]], ssem[k]).wait()

            def sub_chunk(ci, sidx, didx, sidx_n, didx_n, base):
                @pl.when(ci > 0)
                def _():
                    pltpu.make_async_copy(
                        src_hbm.at[pl.ds(base, CSUB)], sidx, isem).wait()
                    pltpu.make_async_copy(
                        dst_hbm.at[pl.ds(base, CSUB)], didx, isem).wait()
                for u in range(CSUB):
                    k = u % NBUF
                    if u < NBUF:
                        @pl.when(ci > 0)
                        def _():
                            wait_scatter(k, didx)
                    else:
                        wait_scatter(k, didx)
                    if u == NBUF:
                        base_n = base + CSUB

                        @pl.when(ci + 1 < nchunks)
                        def _():
                            pltpu.make_async_copy(
                                src_hbm.at[pl.ds(base_n, CSUB)],
                                sidx_n, isem).start()
                            pltpu.make_async_copy(
                                dst_hbm.at[pl.ds(base_n, CSUB)],
                                didx_n, isem).start()
                    pltpu.make_async_copy(
                        tbl.at[sidx.at[u]], bufs[k], gsem[k]).start()
                    if u >= LAG:
                        v = u - LAG
                        k2 = v % NBUF
                        pltpu.make_async_copy(
                            tbl.at[sidx.at[v]], bufs[k2], gsem[k2]).wait()
                        pltpu.make_async_copy(
                            bufs[k2], acc_sh.at[didx.at[v]],
                            ssem[k2]).start(add=True)
                for v in range(CSUB - LAG, CSUB):
                    k2 = v % NBUF
                    pltpu.make_async_copy(
                        tbl.at[sidx.at[v]], bufs[k2], gsem[k2]).wait()
                    pltpu.make_async_copy(
                        bufs[k2], acc_sh.at[didx.at[v]],
                        ssem[k2]).start(add=True)

            def chunk(ci, _):
                base = (first + ci * CH) * SUB

                @pl.when(lax.rem(ci, 2) == 0)
                def _():
                    sub_chunk(ci, sidx_a, didx_a, sidx_b, didx_b, base)

                @pl.when(lax.rem(ci, 2) == 1)
                def _():
                    sub_chunk(ci, sidx_b, didx_b, sidx_a, didx_a, base)

                return 0

            base0 = first * SUB
            pltpu.sync_copy(src_hbm.at[pl.ds(base0, CSUB)], sidx_a)
            pltpu.sync_copy(dst_hbm.at[pl.ds(base0, CSUB)], didx_a)
            lax.fori_loop(0, nchunks, chunk, 0)
            for k in range(NBUF):
                wait_scatter(k, didx_a)
            plsc.subcore_barrier()
            pltpu.sync_copy(acc_sh.at[pl.ds(s * RPT, RPT)],
                            out_hbm.at[out_slot, pl.ds(s * RPT, RPT)])

        if NB > 1:
            for pi in range(NB // NC):
                p = pi * NC + c
                one_block(p, p)
        else:
            one_block(0, c)

    return prop


_prop1 = _make_prop(1)
_prop8 = _make_prop(D // C)


def _norm_body(degp_ref, out_ref):
    d = degp_ref[0:1, :] + degp_ref[1:2, :]
    out_ref[...] = lax.rsqrt(jnp.maximum(d, 1.0))


def _scale_x_body(x_ref, nc_ref, out_ref):
    out_ref[...] = x_ref[...] * nc_ref[...]


def _scale_r_body(r_ref, nc_ref, out_ref):
    n2 = nc_ref[...] * nc_ref[...]
    out_ref[0] = (r_ref[0] + r_ref[1]) * n2


def _scale_q_body(q_ref, nc_ref, out_ref):
    n2 = nc_ref[...] * nc_ref[...]
    for k in range(D // C):
        out_ref[k] = q_ref[k] * n2


def _l1_body(x_ref, r1_ref, r2_ref, nc_ref, w_ref, b_ref, h_ref, hn_ref):
    nc = nc_ref[...]
    p1 = (r1_ref[0] + r1_ref[1]) * nc
    p2 = (r2_ref[0] + r2_ref[1]) * nc
    acc = jnp.dot(x_ref[...], w_ref[0:F, :],
                  preferred_element_type=jnp.float32)
    acc += jnp.dot(p1, w_ref[F:2 * F, :], preferred_element_type=jnp.float32)
    acc += jnp.dot(p2, w_ref[2 * F:3 * F, :],
                   preferred_element_type=jnp.float32)
    h = jnp.maximum(acc + b_ref[...], 0.0)
    h_ref[...] = h
    hn = h * nc
    for k in range(D // C):
        hn_ref[k] = hn[:, k * C:(k + 1) * C]


def _l2_body(h_ref, q1_ref, q2_ref, nc_ref, w_ref, b_ref, cm_ref, sn_ref):
    nc = nc_ref[...]
    acc = jnp.dot(h_ref[...], w_ref[0:D, :], preferred_element_type=jnp.float32)
    for k in range(D // C):
        acc += jnp.dot(q1_ref[k] * nc, w_ref[D + k * C:D + (k + 1) * C, :],
                       preferred_element_type=jnp.float32)
        acc += jnp.dot(q2_ref[k] * nc, w_ref[2 * D + k * C:2 * D + (k + 1) * C, :],
                       preferred_element_type=jnp.float32)
    g = acc + b_ref[...]
    cm_ref[0] = jnp.max(g, axis=0, keepdims=True)
    rn = jnp.sqrt(jnp.sum(g * g, axis=1, keepdims=True))
    sn_ref[0] = jnp.broadcast_to(jnp.sum(rn), (1, 128))


def _head_body(cm_ref, sn_ref, wo1_ref, bo1_ref, wo2_ref, bo2_ref, out_ref):
    colmax = jnp.max(cm_ref[...], axis=0)
    total = jnp.sum(sn_ref[...], axis=0)[:, 0:1]
    factor = (jnp.sqrt(jnp.float32(D)) * jnp.float32(N)) / total
    emb = colmax * factor
    z = jnp.dot(emb, wo1_ref[...], preferred_element_type=jnp.float32)
    z = z + bo1_ref[...]
    z = jnp.where(z > 0, z, 0.01 * z)
    z2 = jnp.dot(z, wo2_ref[...], preferred_element_type=jnp.float32)
    z2 = z2 + bo2_ref[...]
    m = jnp.max(z2, axis=1, keepdims=True)
    lse = jnp.log(jnp.sum(jnp.exp(z2 - m), axis=1, keepdims=True)) + m
    out_ref[...] = z2 - lse


def _row_blocks(*dims):
    def spec(shape, idx_dim):
        def imap(i):
            return tuple(i if d == idx_dim else 0 for d in range(len(shape)))
        return pl.BlockSpec(shape, imap)
    return spec


def kernel(x, edge_index, W1, b1, W2, b2, Wo1, bo1, Wo2, bo2):
    pad = (ERP - ER) * ECH
    pad_src = (jnp.arange(pad, dtype=jnp.int32) * 13) % N
    pad_dst = N + (jnp.arange(pad, dtype=jnp.int32) % (NP - N))
    src2d = jnp.concatenate(
        [edge_index[0].astype(jnp.int32), pad_src]).reshape(ERP, ECH)
    dst2d = jnp.concatenate(
        [edge_index[1].astype(jnp.int32), pad_dst]).reshape(ERP, ECH)
    srcS = src2d.reshape(ERP * SUB, EOP)
    dstS = dst2d.reshape(ERP * SUB, EOP)

    degp = _sc_degree(dst2d)

    norm_row = pl.pallas_call(
        _norm_body,
        out_shape=jax.ShapeDtypeStruct((1, NP), jnp.float32),
    )(degp)
    norm_col = norm_row.reshape(NP, 1)

    nc_spec = pl.BlockSpec((BM, 1), lambda i: (i, 0))

    xn = pl.pallas_call(
        _scale_x_body,
        grid=(GM,),
        in_specs=[pl.BlockSpec((BM, F), lambda i: (i, 0)), nc_spec],
        out_specs=pl.BlockSpec((BM, F), lambda i: (i, 0)),
        out_shape=jax.ShapeDtypeStruct((N, F), jnp.float32),
    )(x, norm_col)

    r1 = _prop1(xn.reshape(1, N, F), srcS, dstS)

    t1 = pl.pallas_call(
        _scale_r_body,
        grid=(GM,),
        in_specs=[pl.BlockSpec((2, BM, F), lambda i: (0, i, 0)), nc_spec],
        out_specs=pl.BlockSpec((1, BM, F), lambda i: (0, i, 0)),
        out_shape=jax.ShapeDtypeStruct((1, N, F), jnp.float32),
    )(r1, norm_col)

    r2 = _prop1(t1, srcS, dstS)

    h, hn = pl.pallas_call(
        _l1_body,
        grid=(GM,),
        in_specs=[
            pl.BlockSpec((BM, F), lambda i: (i, 0)),
            pl.BlockSpec((2, BM, F), lambda i: (0, i, 0)),
            pl.BlockSpec((2, BM, F), lambda i: (0, i, 0)),
            nc_spec,
            pl.BlockSpec((3 * F, D), lambda i: (0, 0)),
            pl.BlockSpec((1, D), lambda i: (0, 0)),
        ],
        out_specs=[
            pl.BlockSpec((BM, D), lambda i: (i, 0)),
            pl.BlockSpec((D // C, BM, C), lambda i: (0, i, 0)),
        ],
        out_shape=[
            jax.ShapeDtypeStruct((N, D), jnp.float32),
            jax.ShapeDtypeStruct((D // C, N, C), jnp.float32),
        ],
    )(x, r1, r2, norm_col, W1, b1.reshape(1, D))

    q1 = _prop8(hn, srcS, dstS)

    t2 = pl.pallas_call(
        _scale_q_body,
        grid=(GM,),
        in_specs=[pl.BlockSpec((D // C, BM, C), lambda i: (0, i, 0)), nc_spec],
        out_specs=pl.BlockSpec((D // C, BM, C), lambda i: (0, i, 0)),
        out_shape=jax.ShapeDtypeStruct((D // C, N, C), jnp.float32),
    )(q1, norm_col)

    q2 = _prop8(t2, srcS, dstS)

    cm, sn = pl.pallas_call(
        _l2_body,
        grid=(GM,),
        in_specs=[
            pl.BlockSpec((BM, D), lambda i: (i, 0)),
            pl.BlockSpec((D // C, BM, C), lambda i: (0, i, 0)),
            pl.BlockSpec((D // C, BM, C), lambda i: (0, i, 0)),
            nc_spec,
            pl.BlockSpec((3 * D, D), lambda i: (0, 0)),
            pl.BlockSpec((1, D), lambda i: (0, 0)),
        ],
        out_specs=[
            pl.BlockSpec((1, 1, D), lambda i: (i, 0, 0)),
            pl.BlockSpec((1, 1, 128), lambda i: (i, 0, 0)),
        ],
        out_shape=[
            jax.ShapeDtypeStruct((GM, 1, D), jnp.float32),
            jax.ShapeDtypeStruct((GM, 1, 128), jnp.float32),
        ],
    )(h, q1, q2, norm_col, W2, b2.reshape(1, D))

    out = pl.pallas_call(
        _head_body,
        out_shape=jax.ShapeDtypeStruct((1, 3), jnp.float32),
    )(cm, sn, Wo1, bo1.reshape(1, 256), Wo2, bo2.reshape(1, 3))
    return out
